# baseline (speedup 1.0000x reference)
import jax
import jax.numpy as jnp
from jax.experimental import pallas as pl

_B, _S = 2, 8192
_HALF, _ROT = 64, 128
_THETA = 1000000.0
_N = _B * _S
_GRID = 16
_ROWS = 8
_BLK = _ROWS * 128


_TWO_OVER_PI = 0.6366197723675814
_C1 = 1.5703125
_C2 = 4.838267948966e-04
_SIGN = -2147483648


def _sincos(x):
    t = x * _TWO_OVER_PI
    qi = (t + 0.5).astype(jnp.int32)
    qf = qi.astype(jnp.float32)
    r = (x - qf * _C1) - qf * _C2
    r2 = r * r
    ps = 8.3333333e-3 + r2 * (-1.9841270e-4)
    ps = -0.16666667 + r2 * ps
    s = r + (r * r2) * ps
    pc = 4.1666667e-2 + r2 * (-1.3888889e-3)
    pc = -0.5 + r2 * pc
    c = 1.0 + r2 * pc
    swap = (qi & 1) == 1
    sin_pre = jnp.where(swap, c, s)
    cos_pre = jnp.where(swap, s, c)
    sin_sign = (qi << 30) & _SIGN
    cos_sign = ((qi + 1) << 30) & _SIGN
    sin_out = jax.lax.bitcast_convert_type(
        jax.lax.bitcast_convert_type(sin_pre, jnp.int32) ^ sin_sign, jnp.float32)
    cos_out = jax.lax.bitcast_convert_type(
        jax.lax.bitcast_convert_type(cos_pre, jnp.int32) ^ cos_sign, jnp.float32)
    return sin_out, cos_out


def _rope_body(pos_ref, cos_ref, sin_ref):
    jcol = jax.lax.broadcasted_iota(jnp.int32, (_HALF, 1), 0).astype(jnp.float32)
    inv_freq_col = 1.0 / (_THETA ** (2.0 * jcol / _ROT))
    for r in range(_ROWS):
        p = pos_ref[0, r, :].astype(jnp.float32)
        pt = jnp.broadcast_to(p.reshape(1, 128), (_HALF, 128))
        ft = pt * inv_freq_col
        sft, cft = _sincos(ft)
        ct = cft.T
        st = sft.T
        cos_ref[pl.ds(r * 128, 128), :] = jnp.concatenate([ct, ct], axis=-1)
        sin_ref[pl.ds(r * 128, 128), :] = jnp.concatenate([st, st], axis=-1)


def kernel(x, position_ids):
    pos = position_ids.reshape(_GRID, _ROWS, 128)
    cos, sin = pl.pallas_call(
        _rope_body,
        grid=(_GRID,),
        in_specs=[pl.BlockSpec((1, _ROWS, 128), lambda i: (i, 0, 0))],
        out_specs=[pl.BlockSpec((_BLK, _ROT), lambda i: (i, 0))] * 2,
        out_shape=[jax.ShapeDtypeStruct((_N, _ROT), jnp.float32)] * 2,
    )(pos)
    dt = x.dtype
    return (cos.reshape(_B, _S, _ROT).astype(dt), sin.reshape(_B, _S, _ROT).astype(dt))


# speedup vs baseline: 1.1781x; 1.1781x over previous
import jax
import jax.numpy as jnp
from jax.experimental import pallas as pl

_B, _S = 2, 8192
_HALF, _ROT = 64, 128
_THETA = 1000000.0
_N = _B * _S
_GRID = 4
_ROWS = 32
_BLK = _ROWS * 128


_TWO_OVER_PI = 0.6366197723675814
_C1 = 1.5703125
_C2 = 4.838267948966e-04
_SIGN = -2147483648


def _sincos(x):
    t = x * _TWO_OVER_PI
    qi = (t + 0.5).astype(jnp.int32)
    qf = qi.astype(jnp.float32)
    r = (x - qf * _C1) - qf * _C2
    r2 = r * r
    ps = 8.3333333e-3 + r2 * (-1.9841270e-4)
    ps = -0.16666667 + r2 * ps
    s = r + (r * r2) * ps
    pc = 4.1666667e-2 + r2 * (-1.3888889e-3)
    pc = -0.5 + r2 * pc
    c = 1.0 + r2 * pc
    swap = (qi & 1) == 1
    sin_pre = jnp.where(swap, c, s)
    cos_pre = jnp.where(swap, s, c)
    sin_sign = (qi << 30) & _SIGN
    cos_sign = ((qi + 1) << 30) & _SIGN
    sin_out = jax.lax.bitcast_convert_type(
        jax.lax.bitcast_convert_type(sin_pre, jnp.int32) ^ sin_sign, jnp.float32)
    cos_out = jax.lax.bitcast_convert_type(
        jax.lax.bitcast_convert_type(cos_pre, jnp.int32) ^ cos_sign, jnp.float32)
    return sin_out, cos_out


def _rope_body(pos_ref, cos_ref, sin_ref):
    jcol = jax.lax.broadcasted_iota(jnp.int32, (_HALF, 1), 0).astype(jnp.float32)
    inv_freq_col = 1.0 / (_THETA ** (2.0 * jcol / _ROT))
    for r in range(_ROWS):
        p = pos_ref[0, r, :].astype(jnp.float32)
        pt = jnp.broadcast_to(p.reshape(1, 128), (_HALF, 128))
        ft = pt * inv_freq_col
        sft, cft = _sincos(ft)
        ct = cft.T
        st = sft.T
        cos_ref[pl.ds(r * 128, 128), :] = jnp.concatenate([ct, ct], axis=-1)
        sin_ref[pl.ds(r * 128, 128), :] = jnp.concatenate([st, st], axis=-1)


def kernel(x, position_ids):
    pos = position_ids.reshape(_GRID, _ROWS, 128)
    cos, sin = pl.pallas_call(
        _rope_body,
        grid=(_GRID,),
        in_specs=[pl.BlockSpec((1, _ROWS, 128), lambda i: (i, 0, 0))],
        out_specs=[pl.BlockSpec((_BLK, _ROT), lambda i: (i, 0))] * 2,
        out_shape=[jax.ShapeDtypeStruct((_N, _ROT), jnp.float32)] * 2,
    )(pos)
    dt = x.dtype
    return (cos.reshape(_B, _S, _ROT).astype(dt), sin.reshape(_B, _S, _ROT).astype(dt))


# speedup vs baseline: 1.2027x; 1.0209x over previous
import jax
import jax.numpy as jnp
from jax.experimental import pallas as pl

_B, _S = 2, 8192
_HALF, _ROT = 64, 128
_THETA = 1000000.0
_N = _B * _S
_GRID = 8
_ROWS = 16
_BLK = _ROWS * 128


_TWO_OVER_PI = 0.6366197723675814
_C1 = 1.5703125
_C2 = 4.838267948966e-04
_SIGN = -2147483648


def _sincos(x):
    t = x * _TWO_OVER_PI
    qi = (t + 0.5).astype(jnp.int32)
    qf = qi.astype(jnp.float32)
    r = (x - qf * _C1) - qf * _C2
    r2 = r * r
    ps = -1.6665558e-1 + r2 * 8.3131310e-3
    s = r + (r * r2) * ps
    pc = 4.1624676e-2 + r2 * (-1.3585908e-3)
    pc = -0.5 + r2 * pc
    c = 1.0 + r2 * pc
    swap = (qi & 1) == 1
    sin_pre = jnp.where(swap, c, s)
    cos_pre = jnp.where(swap, s, c)
    sin_sign = (qi << 30) & _SIGN
    cos_sign = ((qi + 1) << 30) & _SIGN
    sin_out = jax.lax.bitcast_convert_type(
        jax.lax.bitcast_convert_type(sin_pre, jnp.int32) ^ sin_sign, jnp.float32)
    cos_out = jax.lax.bitcast_convert_type(
        jax.lax.bitcast_convert_type(cos_pre, jnp.int32) ^ cos_sign, jnp.float32)
    return sin_out, cos_out


def _rope_body(pos_ref, cos_ref, sin_ref):
    jcol = jax.lax.broadcasted_iota(jnp.int32, (_HALF, 1), 0).astype(jnp.float32)
    inv_freq_col = 1.0 / (_THETA ** (2.0 * jcol / _ROT))
    for r in range(_ROWS):
        p = pos_ref[0, r, :].astype(jnp.float32)
        pt = jnp.broadcast_to(p.reshape(1, 128), (_HALF, 128))
        ft = pt * inv_freq_col
        sft, cft = _sincos(ft)
        ct = cft.T
        st = sft.T
        cos_ref[pl.ds(r * 128, 128), :] = jnp.concatenate([ct, ct], axis=-1)
        sin_ref[pl.ds(r * 128, 128), :] = jnp.concatenate([st, st], axis=-1)


def kernel(x, position_ids):
    pos = position_ids.reshape(_GRID, _ROWS, 128)
    cos, sin = pl.pallas_call(
        _rope_body,
        grid=(_GRID,),
        in_specs=[pl.BlockSpec((1, _ROWS, 128), lambda i: (i, 0, 0))],
        out_specs=[pl.BlockSpec((_BLK, _ROT), lambda i: (i, 0))] * 2,
        out_shape=[jax.ShapeDtypeStruct((_N, _ROT), jnp.float32)] * 2,
    )(pos)
    dt = x.dtype
    return (cos.reshape(_B, _S, _ROT).astype(dt), sin.reshape(_B, _S, _ROT).astype(dt))
